# baseline (speedup 1.0000x reference)
import functools

import jax
import jax.numpy as jnp
from jax import lax
from jax.experimental import pallas as pl
from jax.experimental.pallas import tpu as pltpu
from jax.experimental.pallas import tpu_sc as plsc

BATCH = 16384
EMBED = 64
NCOLS = 1000000
NUM_CORES = 2
NUM_SUBCORES = 16
NUM_WORKERS = NUM_CORES * NUM_SUBCORES
ROWS_PER_W = BATCH // NUM_WORKERS
LANES = 16
PANEL = 512
NPANEL = (NCOLS + PANEL - 1) // PANEL
LAST_PANEL = NPANEL - 1
BASE_PANELS = NPANEL // NUM_WORKERS
EXTRA = NPANEL - BASE_PANELS * NUM_WORKERS
CAP = 1024
ACC_ROWS = 176
FLUSH_AT = 96
NDUMP = 128
EMB_ROWS = BATCH + NDUMP
DUMP = BATCH
NVEC = BATCH // LANES


def _iota():
    return lax.iota(jnp.int32, LANES)


def _compact_range(idx_all, cols, pos, lo, hi):
    def body(g, cnt):
        v = idx_all[pl.ds(g * LANES, LANES)]
        p = g * LANES + _iota()
        m = (v >= lo) & (v < hi)
        plsc.store_compressed(cols.at[pl.ds(cnt, LANES)], v, mask=m)
        plsc.store_compressed(pos.at[pl.ds(cnt, LANES)], p, mask=m)
        npop = plsc.all_reduce_population_count(m)[0]
        return jnp.minimum(cnt + npop, CAP)
    return lax.fori_loop(0, NVEC, body, jnp.int32(0))


def _reset_poslist(poslist):
    for m in range(ACC_ROWS // LANES):
        poslist[pl.ds(m * LANES, LANES)] = jnp.full((LANES,), DUMP, jnp.int32)


def _flush(outrows, poslist, emb_hbm):
    pltpu.sync_copy(outrows, emb_hbm.at[poslist])
    _reset_poslist(poslist)


SUPER = 4
SCAP = 128
CLAMP_COL = (NPANEL - 1) * PANEL - (PANEL - 128)


def _panel_copies(tab_hbm, t, p_lo, buf, sem, issue):
    cbase = jnp.minimum((p_lo + t) * PANEL, CLAMP_COL)
    cb = pl.ds(pl.multiple_of(cbase, 128), PANEL)
    for q in range(EMBED // 8):
        rows = pl.ds(q * 8, 8)
        if issue:
            pltpu.async_copy(tab_hbm.at[rows, cb], buf.at[rows], sem)
        else:
            pltpu.make_async_copy(tab_hbm.at[rows, cb], buf.at[rows],
                                  sem).wait()


def _scan_table(tab_hbm, idx_all, emb_hbm, buf0, buf1, cols, pos,
                scols, spos, loccol, poslist, outrows, sem, p_lo, p_hi):
    cnt = _compact_range(idx_all, cols, pos,
                         p_lo * PANEL, jnp.minimum(p_hi * PANEL, NCOLS))
    nv = (cnt + LANES - 1) // LANES
    nt = p_hi - p_lo
    bufs = (buf0, buf1)

    _panel_copies(tab_hbm, 0, p_lo, buf0, sem, issue=True)

    def super_body(su, h):
        slo = (p_lo + su * SUPER) * PANEL
        shi = slo + SUPER * PANEL

        def sc_body(v, scnt):
            cv = cols[pl.ds(v * LANES, LANES)]
            pv = pos[pl.ds(v * LANES, LANES)]
            valid = (v * LANES + _iota()) < cnt
            m = valid & (cv >= slo) & (cv < shi)
            plsc.store_compressed(scols.at[pl.ds(scnt, LANES)], cv, mask=m)
            plsc.store_compressed(spos.at[pl.ds(scnt, LANES)], pv, mask=m)
            npop = plsc.all_reduce_population_count(m)[0]
            return jnp.minimum(scnt + npop, SCAP)
        scnt = lax.fori_loop(0, nv, sc_body, jnp.int32(0))
        nv2 = (scnt + LANES - 1) // LANES

        for j in range(SUPER):
            t = su * SUPER + j
            mybuf = bufs[j & 1]
            nxtbuf = bufs[(j + 1) & 1]

            @pl.when(t < nt)
            def _():
                _panel_copies(tab_hbm, t, p_lo, mybuf, sem, issue=False)

                @pl.when(t + 1 < nt)
                def _():
                    _panel_copies(tab_hbm, t + 1, p_lo, nxtbuf, sem,
                                  issue=True)

            def make_passes(mybuf, t):
                s = p_lo + t
                pbase = jnp.minimum(s * PANEL, CLAMP_COL)

                def vec_body(v, h):
                    cv = scols[pl.ds(v * LANES, LANES)]
                    pv = spos[pl.ds(v * LANES, LANES)]
                    valid = (v * LANES + _iota()) < scnt
                    m = valid & (cv >= s * PANEL) & (cv < (s + 1) * PANEL)
                    plsc.store_compressed(loccol.at[pl.ds(h, LANES)],
                                          cv - pbase, mask=m)
                    plsc.store_compressed(poslist.at[pl.ds(h, LANES)],
                                          pv, mask=m)
                    npop = plsc.all_reduce_population_count(m)[0]
                    return jnp.minimum(h + npop, ACC_ROWS - LANES)

                def hit_body(hh, _):
                    c = loccol[pl.ds(hh, LANES)][0]
                    for mq in range(EMBED // LANES):
                        kv = mq * LANES + _iota()
                        g = plsc.load_gather(
                            mybuf, [kv, jnp.full((LANES,), c, jnp.int32)])
                        plsc.store_scatter(
                            outrows, [jnp.full((LANES,), hh, jnp.int32),
                                      mq * LANES + _iota()], g)
                    return 0
                return vec_body, hit_body

            vec_body, hit_body = make_passes(mybuf, t)
            live = t < nt
            h_new = lax.fori_loop(0, jnp.where(live, nv2 * 0, 0), vec_body, h)
            lax.fori_loop(h, h_new, hit_body, 0)

            def do_flush():
                _flush(outrows, poslist, emb_hbm)
                return jnp.int32(0)
            h = lax.cond(h_new >= FLUSH_AT, do_flush, lambda: h_new)
        return h

    nsuper = (nt + SUPER - 1) // SUPER
    h_end = lax.fori_loop(0, nsuper, super_body, jnp.int32(0))
    del h_end
    _flush(outrows, poslist, emb_hbm)


def _scan_body(uidx_hbm, iidx_hbm, utab_hbm, itab_hbm, uemb_hbm, iemb_hbm,
               uidx_all, iidx_all, buf0, buf1, cols, pos, scols, spos,
               loccol, poslist, outrows, sem):
    wid = lax.axis_index("s") * NUM_CORES + lax.axis_index("c")
    p_lo = wid * BASE_PANELS + jnp.minimum(wid, EXTRA)
    p_hi = p_lo + BASE_PANELS + jnp.where(wid < EXTRA, 1, 0)

    pltpu.sync_copy(uidx_hbm, uidx_all)
    pltpu.sync_copy(iidx_hbm, iidx_all)
    _reset_poslist(poslist)

    _scan_table(utab_hbm, uidx_all, uemb_hbm, buf0, buf1, cols, pos,
                scols, spos, loccol, poslist, outrows, sem, p_lo, p_hi)
    _scan_table(itab_hbm, iidx_all, iemb_hbm, buf0, buf1, cols, pos,
                scols, spos, loccol, poslist, outrows, sem, p_lo, p_hi)


def _dot_body(uemb_hbm, iemb_hbm, out_hbm, ubuf, ibuf, outv, sem):
    wid = lax.axis_index("s") * NUM_CORES + lax.axis_index("c")
    base = wid * ROWS_PER_W

    def chunk_body(jc, _):
        cbase = base + jc * 128
        cu = pltpu.async_copy(uemb_hbm.at[pl.ds(cbase, 128)], ubuf, sem)
        ci = pltpu.async_copy(iemb_hbm.at[pl.ds(cbase, 128)], ibuf, sem)
        cu.wait()
        ci.wait()

        def group(g, _):
            s = pl.ds(g * LANES, LANES)
            rid = g * LANES + _iota()
            acc = jnp.zeros((LANES,), jnp.float32)
            for k in range(EMBED):
                ck = jnp.full((LANES,), k, jnp.int32)
                u = plsc.load_gather(ubuf, [rid, ck])
                v = plsc.load_gather(ibuf, [rid, ck])
                acc = acc + u * v
            outv[pl.ds(jc * 128 + g * LANES, LANES)] = acc
            return 0
        lax.fori_loop(0, 128 // LANES, group, 0)
        return 0

    lax.fori_loop(0, ROWS_PER_W // 128, chunk_body, 0)
    pltpu.sync_copy(outv, out_hbm.at[pl.ds(base, ROWS_PER_W)])


def kernel(user_indices, item_indices, user_table, item_table):
    mesh = plsc.VectorSubcoreMesh(core_axis_name="c", subcore_axis_name="s")
    params = pltpu.CompilerParams(needs_layout_passes=False)

    scan = functools.partial(
        pl.kernel,
        out_type=(jax.ShapeDtypeStruct((EMB_ROWS, 128), jnp.float32),
                  jax.ShapeDtypeStruct((EMB_ROWS, 128), jnp.float32)),
        mesh=mesh,
        compiler_params=params,
        scratch_types=[
            pltpu.VMEM((BATCH,), jnp.int32),
            pltpu.VMEM((BATCH,), jnp.int32),
            pltpu.VMEM((EMBED, PANEL), jnp.float32),
            pltpu.VMEM((EMBED, PANEL), jnp.float32),
            pltpu.VMEM((CAP + LANES,), jnp.int32),
            pltpu.VMEM((CAP + LANES,), jnp.int32),
            pltpu.VMEM((SCAP + LANES,), jnp.int32),
            pltpu.VMEM((SCAP + LANES,), jnp.int32),
            pltpu.VMEM((ACC_ROWS + LANES,), jnp.int32),
            pltpu.VMEM((ACC_ROWS,), jnp.int32),
            pltpu.VMEM((ACC_ROWS, 128), jnp.float32),
            pltpu.SemaphoreType.DMA,
        ],
    )(_scan_body)

    dot = functools.partial(
        pl.kernel,
        out_type=jax.ShapeDtypeStruct((BATCH,), jnp.float32),
        mesh=mesh,
        compiler_params=params,
        scratch_types=[
            pltpu.VMEM((128, 128), jnp.float32),
            pltpu.VMEM((128, 128), jnp.float32),
            pltpu.VMEM((ROWS_PER_W,), jnp.float32),
            pltpu.SemaphoreType.DMA,
        ],
    )(_dot_body)

    uemb, iemb = scan(user_indices.astype(jnp.int32),
                      item_indices.astype(jnp.int32),
                      user_table.T, item_table.T)
    return dot(uemb, iemb)


# speedup vs baseline: 1.0477x; 1.0477x over previous
import functools

import jax
import jax.numpy as jnp
from jax import lax
from jax.experimental import pallas as pl
from jax.experimental.pallas import tpu as pltpu
from jax.experimental.pallas import tpu_sc as plsc

BATCH = 16384
EMBED = 64
NUM_CORES = 2
NUM_SUBCORES = 16
NUM_WORKERS = NUM_CORES * NUM_SUBCORES
ROWS_PER_W = BATCH // NUM_WORKERS
CHUNK = 128
NCHUNK = ROWS_PER_W // CHUNK
LANES = 16
GROUPS = CHUNK // LANES


def _body(uidx_hbm, iidx_hbm, utab_hbm, itab_hbm, out_hbm,
          uoix, ioix, urows, irows, outv, sem):
    wid = lax.axis_index("s") * NUM_CORES + lax.axis_index("c")
    base = wid * ROWS_PER_W

    def chunk_body(jc, _):
        cbase = base + jc * CHUNK
        pltpu.sync_copy(uidx_hbm.at[pl.ds(cbase, CHUNK)], uoix)
        pltpu.sync_copy(iidx_hbm.at[pl.ds(cbase, CHUNK)], ioix)

        copies = []
        for m in range(GROUPS):
            uvec = uoix[pl.ds(m * LANES, LANES)]
            ivec = ioix[pl.ds(m * LANES, LANES)]
            for l in range(LANES):
                r = m * LANES + l
                copies.append(pltpu.async_copy(
                    utab_hbm.at[uvec[l]], urows.at[r, pl.ds(0, EMBED)], sem))
                copies.append(pltpu.async_copy(
                    itab_hbm.at[ivec[l]], irows.at[r, pl.ds(0, EMBED)], sem))
        for c in copies:
            c.wait()

        def group(g, _):
            rid = g * LANES + lax.iota(jnp.int32, LANES)
            acc = jnp.zeros((LANES,), jnp.float32)
            for k in range(EMBED):
                ck = jnp.full((LANES,), k, jnp.int32)
                u = plsc.load_gather(urows, [rid, ck])
                v = plsc.load_gather(irows, [rid, ck])
                acc = acc + u * v
            outv[pl.ds(jc * CHUNK + g * LANES, LANES)] = acc
            return 0

        lax.fori_loop(0, GROUPS, group, 0)
        return 0

    lax.fori_loop(0, NCHUNK, chunk_body, 0)

    pltpu.sync_copy(outv, out_hbm.at[pl.ds(base, ROWS_PER_W)])


def kernel(user_indices, item_indices, user_table, item_table):
    mesh = plsc.VectorSubcoreMesh(core_axis_name="c", subcore_axis_name="s")
    run = functools.partial(
        pl.kernel,
        out_type=jax.ShapeDtypeStruct((BATCH,), jnp.float32),
        mesh=mesh,
        compiler_params=pltpu.CompilerParams(needs_layout_passes=False),
        scratch_types=[
            pltpu.VMEM((CHUNK,), jnp.int32),
            pltpu.VMEM((CHUNK,), jnp.int32),
            pltpu.VMEM((CHUNK, 2 * EMBED), jnp.float32),
            pltpu.VMEM((CHUNK, 2 * EMBED), jnp.float32),
            pltpu.VMEM((ROWS_PER_W,), jnp.float32),
            pltpu.SemaphoreType.DMA,
        ],
    )(_body)
    return run(user_indices.astype(jnp.int32), item_indices.astype(jnp.int32),
               user_table, item_table)
